# baseline (speedup 1.0000x reference)
import functools

import jax
import jax.numpy as jnp
from jax import lax
from jax.experimental import pallas as pl
from jax.experimental.pallas import tpu as pltpu
from jax.experimental.pallas import tpu_sc as plsc

_ROWS, _COLS = 128, 32768
_N = _ROWS * _COLS
_K = int(_N * 0.9)
_MIN32 = -2147483648

_NTILES = 32
_SHARD = _N // _NTILES
_CHUNK = 16384
_NCHUNK = _SHARD // _CHUNK
_LANES = 16


def _sortable_key(v):
    return jnp.where(v >= 0, v ^ _MIN32, ~v)


_ROWS_PER_TILE = _ROWS // _NTILES
_CHUNKS_PER_ROW = _COLS // _CHUNK


_FLUSH_SMALL = 512


def _reduce_and_store(nbins, hist_v, red_v, dst):
    @plsc.parallel_loop(0, nbins // _LANES, unroll=4)
    def _(c):
        acc = hist_v[pl.ds(c * _LANES, _LANES)]
        for l in range(1, _LANES):
            acc = acc + hist_v[pl.ds(l * nbins + c * _LANES, _LANES)]
        red_v[pl.ds(c * _LANES, _LANES)] = acc
    pltpu.sync_copy(red_v, dst)


def _zero_hist(nbins, hist_v):
    @plsc.parallel_loop(0, nbins, unroll=8)
    def _(z):
        hist_v[pl.ds(z * _LANES, _LANES)] = jnp.zeros((_LANES,), jnp.int32)


def _hist_body(nbins, bucket_shift, prefix_shift, compact, x_hbm, prefix_hbm,
               *refs):
    if compact:
        (out_hbm, cand_hbm, cnt_hbm, buf0_v, buf1_v, pbuf_v, hist_v, red_v,
         cand_v, cntb_v, sem0, sem1) = refs
    else:
        out_hbm, buf0_v, buf1_v, pbuf_v, hist_v, red_v, sem0, sem1 = refs
        cand_hbm = cnt_hbm = cand_v = cntb_v = None
    wid = lax.axis_index("c") * 16 + lax.axis_index("s")
    row0 = wid * _ROWS_PER_TILE

    bufs = (buf0_v, buf1_v)
    sems = (sem0, sem1)

    def _chunk_src(ci):
        row = row0 + ci // _CHUNKS_PER_ROW
        col = (ci % _CHUNKS_PER_ROW) * _CHUNK
        return x_hbm.at[row, pl.ds(col, _CHUNK)]

    descs = [None, None]
    descs[0] = pltpu.async_copy(_chunk_src(0), bufs[0], sems[0])

    _zero_hist(nbins, hist_v)

    if prefix_shift is not None:
        pltpu.sync_copy(prefix_hbm, pbuf_v)
        pvec = pbuf_v[...]
    else:
        pvec = None
    liota = lax.iota(jnp.int32, _LANES)
    lane_base = liota * nbins
    ones = jnp.full((_LANES,), 1, jnp.int32)

    cnts_vec = jnp.zeros((_LANES,), jnp.int32)
    for ci in range(_NCHUNK):
        descs[ci % 2].wait()
        if ci + 1 < _NCHUNK:
            nxt = (ci + 1) % 2
            descs[nxt] = pltpu.async_copy(_chunk_src(ci + 1), bufs[nxt], sems[nxt])
        buf = bufs[ci % 2]

        if not compact:
            @plsc.parallel_loop(0, _CHUNK // _LANES, unroll=8)
            def _(j):
                key = _sortable_key(
                    plsc.bitcast(buf[pl.ds(j * _LANES, _LANES)], jnp.int32))
                bucket = lax.shift_right_logical(key, bucket_shift) & (nbins - 1)
                idx = lane_base + bucket
                if prefix_shift is None:
                    plsc.addupdate_scatter(hist_v, [idx], ones)
                else:
                    m = lax.shift_right_logical(key, prefix_shift) == pvec
                    plsc.addupdate_scatter(hist_v, [idx], ones, mask=m)
        else:
            @plsc.parallel_loop(0, _CHUNK // _LANES, unroll=8,
                                carry=jnp.int32(0))
            def cnt(j, cnt):
                key = _sortable_key(
                    plsc.bitcast(buf[pl.ds(j * _LANES, _LANES)], jnp.int32))
                bucket = lax.shift_right_logical(key, bucket_shift) & (nbins - 1)
                m = lax.shift_right_logical(key, prefix_shift) == pvec
                plsc.addupdate_scatter(hist_v, [lane_base + bucket], ones,
                                       mask=m)
                plsc.store_compressed(cand_v.at[pl.ds(cnt, _LANES)], key,
                                      mask=m)
                return cnt + jnp.sum(m.astype(jnp.int32))

            seg = wid * _SHARD + ci * _CHUNK

            @pl.when(jnp.logical_and(cnt > 0, cnt <= _FLUSH_SMALL))
            def _():
                pltpu.sync_copy(cand_v.at[pl.ds(0, _FLUSH_SMALL)],
                                cand_hbm.at[pl.ds(seg, _FLUSH_SMALL)])

            @pl.when(cnt > _FLUSH_SMALL)
            def _():
                pltpu.sync_copy(cand_v.at[pl.ds(0, _CHUNK)],
                                cand_hbm.at[pl.ds(seg, _CHUNK)])

            cnts_vec = jnp.where(liota == ci, cnt, cnts_vec)

    if compact:
        cntb_v[...] = cnts_vec
        pltpu.sync_copy(cntb_v, cnt_hbm.at[wid])
    _reduce_and_store(nbins, hist_v, red_v, out_hbm.at[wid])


def _cand_hist_body(nbins, prefix_shift, cand_hbm, cnt_hbm, prefix_hbm,
                    out_hbm, buf_v, pbuf_v, cntb_v, hist_v, red_v):
    wid = lax.axis_index("c") * 16 + lax.axis_index("s")

    pltpu.sync_copy(cnt_hbm.at[wid], cntb_v)
    _zero_hist(nbins, hist_v)
    pltpu.sync_copy(prefix_hbm, pbuf_v)
    pvec = pbuf_v[...]
    cnts = cntb_v[...]
    liota = lax.iota(jnp.int32, _LANES)
    lane_base = liota * nbins
    ones = jnp.full((_LANES,), 1, jnp.int32)

    for ci in range(_NCHUNK):
        cnt = cnts[ci]
        seg = wid * _SHARD + ci * _CHUNK

        @pl.when(jnp.logical_and(cnt > 0, cnt <= _FLUSH_SMALL))
        def _():
            pltpu.sync_copy(cand_hbm.at[pl.ds(seg, _FLUSH_SMALL)],
                            buf_v.at[pl.ds(0, _FLUSH_SMALL)])

        @pl.when(cnt > _FLUSH_SMALL)
        def _():
            pltpu.sync_copy(cand_hbm.at[pl.ds(seg, _CHUNK)], buf_v)

        def _vreg(j, carry):
            key = buf_v[pl.ds(j * _LANES, _LANES)]
            bucket = key & (nbins - 1)
            valid = (j * _LANES + liota) < cnt
            m = jnp.logical_and(
                lax.shift_right_logical(key, prefix_shift) == pvec, valid)
            plsc.addupdate_scatter(hist_v, [lane_base + bucket], ones, mask=m)
            return carry

        lax.fori_loop(0, lax.div(cnt + (_LANES - 1), _LANES), _vreg,
                      jnp.int32(0))

    _reduce_and_store(nbins, hist_v, red_v, out_hbm.at[wid])


def _make_hist_kernel(nbins, bucket_shift, prefix_shift, compact=False):
    mesh = plsc.VectorSubcoreMesh(core_axis_name="c", subcore_axis_name="s")
    if compact:
        out_type = [
            jax.ShapeDtypeStruct((_NTILES, nbins), jnp.int32),
            jax.ShapeDtypeStruct((_N,), jnp.int32),
            jax.ShapeDtypeStruct((_NTILES, _LANES), jnp.int32),
        ]
        extra = [pltpu.VMEM((_CHUNK + _LANES,), jnp.int32),
                 pltpu.VMEM((_LANES,), jnp.int32)]
    else:
        out_type = jax.ShapeDtypeStruct((_NTILES, nbins), jnp.int32)
        extra = []
    return functools.partial(
        pl.kernel,
        out_type=out_type,
        mesh=mesh,
        scratch_types=[
            pltpu.VMEM((_CHUNK,), jnp.float32),
            pltpu.VMEM((_CHUNK,), jnp.float32),
            pltpu.VMEM((_LANES,), jnp.int32),
            pltpu.VMEM((nbins * _LANES,), jnp.int32),
            pltpu.VMEM((nbins,), jnp.int32),
        ] + extra + [
            pltpu.SemaphoreType.DMA,
            pltpu.SemaphoreType.DMA,
        ],
        compiler_params=pltpu.CompilerParams(needs_layout_passes=False),
    )(functools.partial(_hist_body, nbins, bucket_shift, prefix_shift,
                        compact))


def _make_cand_hist_kernel(nbins, prefix_shift):
    mesh = plsc.VectorSubcoreMesh(core_axis_name="c", subcore_axis_name="s")
    return functools.partial(
        pl.kernel,
        out_type=jax.ShapeDtypeStruct((_NTILES, nbins), jnp.int32),
        mesh=mesh,
        scratch_types=[
            pltpu.VMEM((_CHUNK,), jnp.int32),
            pltpu.VMEM((_LANES,), jnp.int32),
            pltpu.VMEM((_LANES,), jnp.int32),
            pltpu.VMEM((nbins * _LANES,), jnp.int32),
            pltpu.VMEM((nbins,), jnp.int32),
        ],
        compiler_params=pltpu.CompilerParams(needs_layout_passes=False),
    )(functools.partial(_cand_hist_body, nbins, prefix_shift))


_hist_p1 = _make_hist_kernel(4096, 20, None)
_hist_p2 = _make_hist_kernel(4096, 8, 20, compact=True)
_hist_p3 = _make_cand_hist_kernel(256, 8)


def _pick_bin(tile_hists, nbins, k):
    del nbins
    hist = tile_hists.sum(axis=0)
    suffix = jnp.cumsum(hist[::-1])[::-1]
    b = jnp.sum((suffix >= k).astype(jnp.int32)) - 1
    k_next = k - (suffix[b] - hist[b])
    return b, k_next


def _mask_body(t_ref, x_ref, out_ref):
    ts = t_ref[0, 0] ^ _MIN32
    ubits = jnp.where(ts >= 0, ts, _MIN32 | (~ts))
    t = lax.bitcast_convert_type(ubits, jnp.float32)
    xv = x_ref[...]
    m = (xv >= t).astype(jnp.float32)
    out_ref[...] = xv + (m - xv)


def kernel(x):
    zeros16 = jnp.zeros((_LANES,), jnp.int32)

    h1 = _hist_p1(x, zeros16)
    b1, k2 = _pick_bin(h1, 4096, _K)
    h2, cand, cnts = _hist_p2(x, zeros16 + b1)
    b2, k3 = _pick_bin(h2, 4096, k2)
    h3 = _hist_p3(cand, cnts, zeros16 + ((b1 << 12) | b2))
    b3, _ = _pick_bin(h3, 256, k3)

    t_key = ((b1 << 20) | (b2 << 8) | b3).astype(jnp.int32).reshape(1, 1)

    block_rows = 32
    grid = _ROWS // block_rows
    out = pl.pallas_call(
        _mask_body,
        grid=(grid,),
        in_specs=[
            pl.BlockSpec(memory_space=pltpu.SMEM),
            pl.BlockSpec((block_rows, _COLS), lambda i: (i, 0)),
        ],
        out_specs=pl.BlockSpec((block_rows, _COLS), lambda i: (i, 0)),
        out_shape=jax.ShapeDtypeStruct((_ROWS, _COLS), jnp.float32),
    )(t_key, x)
    return out


# speedup vs baseline: 1.0397x; 1.0397x over previous
import functools

import jax
import jax.numpy as jnp
from jax import lax
from jax.experimental import pallas as pl
from jax.experimental.pallas import tpu as pltpu
from jax.experimental.pallas import tpu_sc as plsc

_ROWS, _COLS = 128, 32768
_N = _ROWS * _COLS
_K = int(_N * 0.9)
_MIN32 = -2147483648

_NTILES = 32
_SHARD = _N // _NTILES
_CHUNK = 16384
_NCHUNK = _SHARD // _CHUNK
_LANES = 16


def _sortable_key(v):
    return jnp.where(v >= 0, v ^ _MIN32, ~v)


_ROWS_PER_TILE = _ROWS // _NTILES
_CHUNKS_PER_ROW = _COLS // _CHUNK


_FLUSH_SMALL = 512


def _reduce_and_store(nbins, hist_v, red_v, dst):
    @plsc.parallel_loop(0, nbins // _LANES, unroll=4)
    def _(c):
        acc = hist_v[pl.ds(c * _LANES, _LANES)]
        for l in range(1, _LANES):
            acc = acc + hist_v[pl.ds(l * nbins + c * _LANES, _LANES)]
        red_v[pl.ds(c * _LANES, _LANES)] = acc
    pltpu.sync_copy(red_v, dst)


def _zero_hist(nbins, hist_v):
    @plsc.parallel_loop(0, nbins, unroll=8)
    def _(z):
        hist_v[pl.ds(z * _LANES, _LANES)] = jnp.zeros((_LANES,), jnp.int32)


def _hist_body(nbins, bucket_shift, prefix_shift, compact, x_hbm, prefix_hbm,
               *refs):
    if compact:
        (out_hbm, cand_hbm, cnt_hbm, buf0_v, buf1_v, pbuf_v, hist_v, red_v,
         cand_v, cntb_v, sem0, sem1) = refs
    else:
        out_hbm, buf0_v, buf1_v, pbuf_v, hist_v, red_v, sem0, sem1 = refs
        cand_hbm = cnt_hbm = cand_v = cntb_v = None
    wid = lax.axis_index("c") * 16 + lax.axis_index("s")
    row0 = wid * _ROWS_PER_TILE

    bufs = (buf0_v, buf1_v)
    sems = (sem0, sem1)

    def _chunk_src(ci):
        row = row0 + ci // _CHUNKS_PER_ROW
        col = (ci % _CHUNKS_PER_ROW) * _CHUNK
        return x_hbm.at[row, pl.ds(col, _CHUNK)]

    descs = [None, None]
    descs[0] = pltpu.async_copy(_chunk_src(0), bufs[0], sems[0])

    _zero_hist(nbins, hist_v)

    if prefix_shift is not None:
        pltpu.sync_copy(prefix_hbm, pbuf_v)
        pvec = pbuf_v[...]
    else:
        pvec = None
    liota = lax.iota(jnp.int32, _LANES)
    lane_base = liota * nbins
    ones = jnp.full((_LANES,), 1, jnp.int32)

    cnts_vec = jnp.zeros((_LANES,), jnp.int32)
    for ci in range(_NCHUNK):
        descs[ci % 2].wait()
        if ci + 1 < _NCHUNK:
            nxt = (ci + 1) % 2
            descs[nxt] = pltpu.async_copy(_chunk_src(ci + 1), bufs[nxt], sems[nxt])
        buf = bufs[ci % 2]

        if not compact:
            @plsc.parallel_loop(0, _CHUNK // _LANES, unroll=8)
            def _(j):
                key = _sortable_key(
                    plsc.bitcast(buf[pl.ds(j * _LANES, _LANES)], jnp.int32))
                bucket = lax.shift_right_logical(key, bucket_shift) & (nbins - 1)
                idx = lane_base + bucket
                if prefix_shift is None:
                    plsc.addupdate_scatter(hist_v, [idx], ones)
                else:
                    m = lax.shift_right_logical(key, prefix_shift) == pvec
                    plsc.addupdate_scatter(hist_v, [idx], ones, mask=m)
        else:
            @plsc.parallel_loop(0, _CHUNK // _LANES, unroll=8,
                                carry=jnp.int32(0))
            def cnt(j, cnt):
                key = _sortable_key(
                    plsc.bitcast(buf[pl.ds(j * _LANES, _LANES)], jnp.int32))
                bucket = lax.shift_right_logical(key, bucket_shift) & (nbins - 1)
                m = lax.shift_right_logical(key, prefix_shift) == pvec
                plsc.addupdate_scatter(hist_v, [lane_base + bucket], ones,
                                       mask=m)
                plsc.store_compressed(cand_v.at[pl.ds(cnt, _LANES)], key,
                                      mask=m)
                return cnt + jnp.sum(m.astype(jnp.int32))

            seg = wid * _SHARD + ci * _CHUNK

            @pl.when(jnp.logical_and(cnt > 0, cnt <= _FLUSH_SMALL))
            def _():
                pltpu.sync_copy(cand_v.at[pl.ds(0, _FLUSH_SMALL)],
                                cand_hbm.at[pl.ds(seg, _FLUSH_SMALL)])

            @pl.when(cnt > _FLUSH_SMALL)
            def _():
                pltpu.sync_copy(cand_v.at[pl.ds(0, _CHUNK)],
                                cand_hbm.at[pl.ds(seg, _CHUNK)])

            cnts_vec = jnp.where(liota == ci, cnt, cnts_vec)

    if compact:
        cntb_v[...] = cnts_vec
        pltpu.sync_copy(cntb_v, cnt_hbm.at[wid])
    _reduce_and_store(nbins, hist_v, red_v, out_hbm.at[wid])


def _cand_hist_body(nbins, prefix_shift, cand_hbm, cnt_hbm, prefix_hbm,
                    out_hbm, buf_v, stage_v, pbuf_v, cntb_v, hist_v, red_v,
                    *sems):
    wid = lax.axis_index("c") * 16 + lax.axis_index("s")

    descs = [
        pltpu.async_copy(
            cand_hbm.at[pl.ds(wid * _SHARD + ci * _CHUNK, _FLUSH_SMALL)],
            stage_v.at[pl.ds(ci * _FLUSH_SMALL, _FLUSH_SMALL)], sems[ci])
        for ci in range(_NCHUNK)
    ]
    pltpu.sync_copy(cnt_hbm.at[wid], cntb_v)
    _zero_hist(nbins, hist_v)
    pltpu.sync_copy(prefix_hbm, pbuf_v)
    pvec = pbuf_v[...]
    cnts = cntb_v[...]
    liota = lax.iota(jnp.int32, _LANES)
    lane_base = liota * nbins
    ones = jnp.full((_LANES,), 1, jnp.int32)

    def _seg_loop(src_v, src_off, cnt):
        def _vreg(j, carry):
            key = src_v[pl.ds(src_off + j * _LANES, _LANES)]
            bucket = key & (nbins - 1)
            valid = (j * _LANES + liota) < cnt
            m = jnp.logical_and(
                lax.shift_right_logical(key, prefix_shift) == pvec, valid)
            plsc.addupdate_scatter(hist_v, [lane_base + bucket], ones, mask=m)
            return carry

        lax.fori_loop(0, lax.div(cnt + (_LANES - 1), _LANES), _vreg,
                      jnp.int32(0))

    for ci in range(_NCHUNK):
        cnt = cnts[ci]
        descs[ci].wait()

        @pl.when(cnt <= _FLUSH_SMALL)
        def _():
            _seg_loop(stage_v, ci * _FLUSH_SMALL, cnt)

        @pl.when(cnt > _FLUSH_SMALL)
        def _():
            pltpu.sync_copy(
                cand_hbm.at[pl.ds(wid * _SHARD + ci * _CHUNK, _CHUNK)], buf_v)
            _seg_loop(buf_v, 0, cnt)

    _reduce_and_store(nbins, hist_v, red_v, out_hbm.at[wid])


def _make_hist_kernel(nbins, bucket_shift, prefix_shift, compact=False):
    mesh = plsc.VectorSubcoreMesh(core_axis_name="c", subcore_axis_name="s")
    if compact:
        out_type = [
            jax.ShapeDtypeStruct((_NTILES, nbins), jnp.int32),
            jax.ShapeDtypeStruct((_N,), jnp.int32),
            jax.ShapeDtypeStruct((_NTILES, _LANES), jnp.int32),
        ]
        extra = [pltpu.VMEM((_CHUNK + _LANES,), jnp.int32),
                 pltpu.VMEM((_LANES,), jnp.int32)]
    else:
        out_type = jax.ShapeDtypeStruct((_NTILES, nbins), jnp.int32)
        extra = []
    return functools.partial(
        pl.kernel,
        out_type=out_type,
        mesh=mesh,
        scratch_types=[
            pltpu.VMEM((_CHUNK,), jnp.float32),
            pltpu.VMEM((_CHUNK,), jnp.float32),
            pltpu.VMEM((_LANES,), jnp.int32),
            pltpu.VMEM((nbins * _LANES,), jnp.int32),
            pltpu.VMEM((nbins,), jnp.int32),
        ] + extra + [
            pltpu.SemaphoreType.DMA,
            pltpu.SemaphoreType.DMA,
        ],
        compiler_params=pltpu.CompilerParams(needs_layout_passes=False),
    )(functools.partial(_hist_body, nbins, bucket_shift, prefix_shift,
                        compact))


def _make_cand_hist_kernel(nbins, prefix_shift):
    mesh = plsc.VectorSubcoreMesh(core_axis_name="c", subcore_axis_name="s")
    return functools.partial(
        pl.kernel,
        out_type=jax.ShapeDtypeStruct((_NTILES, nbins), jnp.int32),
        mesh=mesh,
        scratch_types=[
            pltpu.VMEM((_CHUNK,), jnp.int32),
            pltpu.VMEM((_NCHUNK * _FLUSH_SMALL,), jnp.int32),
            pltpu.VMEM((_LANES,), jnp.int32),
            pltpu.VMEM((_LANES,), jnp.int32),
            pltpu.VMEM((nbins * _LANES,), jnp.int32),
            pltpu.VMEM((nbins,), jnp.int32),
        ] + [pltpu.SemaphoreType.DMA] * _NCHUNK,
        compiler_params=pltpu.CompilerParams(needs_layout_passes=False),
    )(functools.partial(_cand_hist_body, nbins, prefix_shift))


_hist_p1 = _make_hist_kernel(4096, 20, None)
_hist_p2 = _make_hist_kernel(4096, 8, 20, compact=True)
_hist_p3 = _make_cand_hist_kernel(256, 8)


def _pick_bin(tile_hists, nbins, k):
    del nbins
    hist = tile_hists.sum(axis=0)
    suffix = jnp.cumsum(hist[::-1])[::-1]
    b = jnp.sum((suffix >= k).astype(jnp.int32)) - 1
    k_next = k - (suffix[b] - hist[b])
    return b, k_next


def _mask_body(t_ref, x_ref, out_ref):
    ts = t_ref[0, 0] ^ _MIN32
    ubits = jnp.where(ts >= 0, ts, _MIN32 | (~ts))
    t = lax.bitcast_convert_type(ubits, jnp.float32)
    xv = x_ref[...]
    m = (xv >= t).astype(jnp.float32)
    out_ref[...] = xv + (m - xv)


def kernel(x):
    zeros16 = jnp.zeros((_LANES,), jnp.int32)

    h1 = _hist_p1(x, zeros16)
    b1, k2 = _pick_bin(h1, 4096, _K)
    h2, cand, cnts = _hist_p2(x, zeros16 + b1)
    b2, k3 = _pick_bin(h2, 4096, k2)
    h3 = _hist_p3(cand, cnts, zeros16 + ((b1 << 12) | b2))
    b3, _ = _pick_bin(h3, 256, k3)

    t_key = ((b1 << 20) | (b2 << 8) | b3).astype(jnp.int32).reshape(1, 1)

    block_rows = 32
    grid = _ROWS // block_rows
    out = pl.pallas_call(
        _mask_body,
        grid=(grid,),
        in_specs=[
            pl.BlockSpec(memory_space=pltpu.SMEM),
            pl.BlockSpec((block_rows, _COLS), lambda i: (i, 0)),
        ],
        out_specs=pl.BlockSpec((block_rows, _COLS), lambda i: (i, 0)),
        out_shape=jax.ShapeDtypeStruct((_ROWS, _COLS), jnp.float32),
    )(t_key, x)
    return out


# speedup vs baseline: 1.0398x; 1.0001x over previous
import functools

import jax
import jax.numpy as jnp
from jax import lax
from jax.experimental import pallas as pl
from jax.experimental.pallas import tpu as pltpu
from jax.experimental.pallas import tpu_sc as plsc

_ROWS, _COLS = 128, 32768
_N = _ROWS * _COLS
_K = int(_N * 0.9)
_MIN32 = -2147483648

_NTILES = 32
_SHARD = _N // _NTILES
_CHUNK = 16384
_NCHUNK = _SHARD // _CHUNK
_LANES = 16


def _sortable_key(v):
    return jnp.where(v >= 0, v ^ _MIN32, ~v)


_ROWS_PER_TILE = _ROWS // _NTILES
_CHUNKS_PER_ROW = _COLS // _CHUNK


_FLUSH_SMALL = 512


def _reduce_and_store(nbins, hist_v, red_v, dst):
    @plsc.parallel_loop(0, nbins // _LANES, unroll=4)
    def _(c):
        acc = hist_v[pl.ds(c * _LANES, _LANES)]
        for l in range(1, _LANES):
            acc = acc + hist_v[pl.ds(l * nbins + c * _LANES, _LANES)]
        red_v[pl.ds(c * _LANES, _LANES)] = acc
    pltpu.sync_copy(red_v, dst)


def _zero_hist(nbins, hist_v):
    @plsc.parallel_loop(0, nbins, unroll=8)
    def _(z):
        hist_v[pl.ds(z * _LANES, _LANES)] = jnp.zeros((_LANES,), jnp.int32)


def _hist_body(nbins, bucket_shift, prefix_shift, compact, x_hbm, prefix_hbm,
               *refs):
    if compact:
        (out_hbm, cand_hbm, cnt_hbm, buf0_v, buf1_v, pbuf_v, hist_v, red_v,
         cand_v, cntb_v, sem0, sem1) = refs
    else:
        out_hbm, buf0_v, buf1_v, pbuf_v, hist_v, red_v, sem0, sem1 = refs
        cand_hbm = cnt_hbm = cand_v = cntb_v = None
    wid = lax.axis_index("c") * 16 + lax.axis_index("s")
    row0 = wid * _ROWS_PER_TILE

    bufs = (buf0_v, buf1_v)
    sems = (sem0, sem1)

    def _chunk_src(ci):
        row = row0 + ci // _CHUNKS_PER_ROW
        col = (ci % _CHUNKS_PER_ROW) * _CHUNK
        return x_hbm.at[row, pl.ds(col, _CHUNK)]

    descs = [None, None]
    descs[0] = pltpu.async_copy(_chunk_src(0), bufs[0], sems[0])

    _zero_hist(nbins, hist_v)

    if prefix_shift is not None:
        pltpu.sync_copy(prefix_hbm, pbuf_v)
        pvec = pbuf_v[...]
    else:
        pvec = None
    liota = lax.iota(jnp.int32, _LANES)
    lane_base = liota * nbins
    ones = jnp.full((_LANES,), 1, jnp.int32)

    cnts_vec = jnp.zeros((_LANES,), jnp.int32)
    for ci in range(_NCHUNK):
        descs[ci % 2].wait()
        if ci + 1 < _NCHUNK:
            nxt = (ci + 1) % 2
            descs[nxt] = pltpu.async_copy(_chunk_src(ci + 1), bufs[nxt], sems[nxt])
        buf = bufs[ci % 2]

        if not compact:
            @plsc.parallel_loop(0, _CHUNK // _LANES, unroll=8)
            def _(j):
                key = _sortable_key(
                    plsc.bitcast(buf[pl.ds(j * _LANES, _LANES)], jnp.int32))
                bucket = lax.shift_right_logical(key, bucket_shift) & (nbins - 1)
                idx = lane_base + bucket
                if prefix_shift is None:
                    plsc.addupdate_scatter(hist_v, [idx], ones)
                else:
                    m = lax.shift_right_logical(key, prefix_shift) == pvec
                    plsc.addupdate_scatter(hist_v, [idx], ones, mask=m)
        else:
            @plsc.parallel_loop(0, _CHUNK // _LANES, unroll=8,
                                carry=jnp.int32(0))
            def cnt(j, cnt):
                key = _sortable_key(
                    plsc.bitcast(buf[pl.ds(j * _LANES, _LANES)], jnp.int32))
                bucket = lax.shift_right_logical(key, bucket_shift) & (nbins - 1)
                m = lax.shift_right_logical(key, prefix_shift) == pvec
                plsc.addupdate_scatter(hist_v, [lane_base + bucket], ones,
                                       mask=m)
                plsc.store_compressed(cand_v.at[pl.ds(cnt, _LANES)], key,
                                      mask=m)
                return cnt + plsc.all_reduce_population_count(m)[0]

            seg = wid * _SHARD + ci * _CHUNK

            @pl.when(jnp.logical_and(cnt > 0, cnt <= _FLUSH_SMALL))
            def _():
                pltpu.sync_copy(cand_v.at[pl.ds(0, _FLUSH_SMALL)],
                                cand_hbm.at[pl.ds(seg, _FLUSH_SMALL)])

            @pl.when(cnt > _FLUSH_SMALL)
            def _():
                pltpu.sync_copy(cand_v.at[pl.ds(0, _CHUNK)],
                                cand_hbm.at[pl.ds(seg, _CHUNK)])

            cnts_vec = jnp.where(liota == ci, cnt, cnts_vec)

    if compact:
        cntb_v[...] = cnts_vec
        pltpu.sync_copy(cntb_v, cnt_hbm.at[wid])
    _reduce_and_store(nbins, hist_v, red_v, out_hbm.at[wid])


def _cand_hist_body(nbins, prefix_shift, cand_hbm, cnt_hbm, prefix_hbm,
                    out_hbm, buf_v, stage_v, pbuf_v, cntb_v, hist_v, red_v,
                    *sems):
    wid = lax.axis_index("c") * 16 + lax.axis_index("s")

    descs = [
        pltpu.async_copy(
            cand_hbm.at[pl.ds(wid * _SHARD + ci * _CHUNK, _FLUSH_SMALL)],
            stage_v.at[pl.ds(ci * _FLUSH_SMALL, _FLUSH_SMALL)], sems[ci])
        for ci in range(_NCHUNK)
    ]
    pltpu.sync_copy(cnt_hbm.at[wid], cntb_v)
    _zero_hist(nbins, hist_v)
    pltpu.sync_copy(prefix_hbm, pbuf_v)
    pvec = pbuf_v[...]
    cnts = cntb_v[...]
    liota = lax.iota(jnp.int32, _LANES)
    lane_base = liota * nbins
    ones = jnp.full((_LANES,), 1, jnp.int32)

    def _seg_loop(src_v, src_off, cnt):
        def _vreg(j, carry):
            key = src_v[pl.ds(src_off + j * _LANES, _LANES)]
            bucket = key & (nbins - 1)
            valid = (j * _LANES + liota) < cnt
            m = jnp.logical_and(
                lax.shift_right_logical(key, prefix_shift) == pvec, valid)
            plsc.addupdate_scatter(hist_v, [lane_base + bucket], ones, mask=m)
            return carry

        lax.fori_loop(0, lax.div(cnt + (_LANES - 1), _LANES), _vreg,
                      jnp.int32(0))

    for ci in range(_NCHUNK):
        cnt = cnts[ci]
        descs[ci].wait()

        @pl.when(cnt <= _FLUSH_SMALL)
        def _():
            _seg_loop(stage_v, ci * _FLUSH_SMALL, cnt)

        @pl.when(cnt > _FLUSH_SMALL)
        def _():
            pltpu.sync_copy(
                cand_hbm.at[pl.ds(wid * _SHARD + ci * _CHUNK, _CHUNK)], buf_v)
            _seg_loop(buf_v, 0, cnt)

    _reduce_and_store(nbins, hist_v, red_v, out_hbm.at[wid])


def _make_hist_kernel(nbins, bucket_shift, prefix_shift, compact=False):
    mesh = plsc.VectorSubcoreMesh(core_axis_name="c", subcore_axis_name="s")
    if compact:
        out_type = [
            jax.ShapeDtypeStruct((_NTILES, nbins), jnp.int32),
            jax.ShapeDtypeStruct((_N,), jnp.int32),
            jax.ShapeDtypeStruct((_NTILES, _LANES), jnp.int32),
        ]
        extra = [pltpu.VMEM((_CHUNK + _LANES,), jnp.int32),
                 pltpu.VMEM((_LANES,), jnp.int32)]
    else:
        out_type = jax.ShapeDtypeStruct((_NTILES, nbins), jnp.int32)
        extra = []
    return functools.partial(
        pl.kernel,
        out_type=out_type,
        mesh=mesh,
        scratch_types=[
            pltpu.VMEM((_CHUNK,), jnp.float32),
            pltpu.VMEM((_CHUNK,), jnp.float32),
            pltpu.VMEM((_LANES,), jnp.int32),
            pltpu.VMEM((nbins * _LANES,), jnp.int32),
            pltpu.VMEM((nbins,), jnp.int32),
        ] + extra + [
            pltpu.SemaphoreType.DMA,
            pltpu.SemaphoreType.DMA,
        ],
        compiler_params=pltpu.CompilerParams(needs_layout_passes=False),
    )(functools.partial(_hist_body, nbins, bucket_shift, prefix_shift,
                        compact))


def _make_cand_hist_kernel(nbins, prefix_shift):
    mesh = plsc.VectorSubcoreMesh(core_axis_name="c", subcore_axis_name="s")
    return functools.partial(
        pl.kernel,
        out_type=jax.ShapeDtypeStruct((_NTILES, nbins), jnp.int32),
        mesh=mesh,
        scratch_types=[
            pltpu.VMEM((_CHUNK,), jnp.int32),
            pltpu.VMEM((_NCHUNK * _FLUSH_SMALL,), jnp.int32),
            pltpu.VMEM((_LANES,), jnp.int32),
            pltpu.VMEM((_LANES,), jnp.int32),
            pltpu.VMEM((nbins * _LANES,), jnp.int32),
            pltpu.VMEM((nbins,), jnp.int32),
        ] + [pltpu.SemaphoreType.DMA] * _NCHUNK,
        compiler_params=pltpu.CompilerParams(needs_layout_passes=False),
    )(functools.partial(_cand_hist_body, nbins, prefix_shift))


_hist_p1 = _make_hist_kernel(4096, 20, None)
_hist_p2 = _make_hist_kernel(4096, 8, 20, compact=True)
_hist_p3 = _make_cand_hist_kernel(256, 8)


def _pick_bin(tile_hists, nbins, k):
    del nbins
    hist = tile_hists.sum(axis=0)
    suffix = jnp.cumsum(hist[::-1])[::-1]
    b = jnp.sum((suffix >= k).astype(jnp.int32)) - 1
    k_next = k - (suffix[b] - hist[b])
    return b, k_next


def _mask_body(t_ref, x_ref, out_ref):
    ts = t_ref[0, 0] ^ _MIN32
    ubits = jnp.where(ts >= 0, ts, _MIN32 | (~ts))
    t = lax.bitcast_convert_type(ubits, jnp.float32)
    xv = x_ref[...]
    m = (xv >= t).astype(jnp.float32)
    out_ref[...] = xv + (m - xv)


def kernel(x):
    zeros16 = jnp.zeros((_LANES,), jnp.int32)

    h1 = _hist_p1(x, zeros16)
    b1, k2 = _pick_bin(h1, 4096, _K)
    h2, cand, cnts = _hist_p2(x, zeros16 + b1)
    b2, k3 = _pick_bin(h2, 4096, k2)
    h3 = _hist_p3(cand, cnts, zeros16 + ((b1 << 12) | b2))
    b3, _ = _pick_bin(h3, 256, k3)

    t_key = ((b1 << 20) | (b2 << 8) | b3).astype(jnp.int32).reshape(1, 1)

    block_rows = 32
    grid = _ROWS // block_rows
    out = pl.pallas_call(
        _mask_body,
        grid=(grid,),
        in_specs=[
            pl.BlockSpec(memory_space=pltpu.SMEM),
            pl.BlockSpec((block_rows, _COLS), lambda i: (i, 0)),
        ],
        out_specs=pl.BlockSpec((block_rows, _COLS), lambda i: (i, 0)),
        out_shape=jax.ShapeDtypeStruct((_ROWS, _COLS), jnp.float32),
    )(t_key, x)
    return out


# speedup vs baseline: 1.0869x; 1.0454x over previous
import functools

import jax
import jax.numpy as jnp
from jax import lax
from jax.experimental import pallas as pl
from jax.experimental.pallas import tpu as pltpu
from jax.experimental.pallas import tpu_sc as plsc

_ROWS, _COLS = 128, 32768
_N = _ROWS * _COLS
_K = int(_N * 0.9)
_MIN32 = -2147483648

_NTILES = 32
_SHARD = _N // _NTILES
_CHUNK = 16384
_NCHUNK = _SHARD // _CHUNK
_LANES = 16


def _sortable_key(v):
    return v ^ (lax.shift_right_arithmetic(v, 31) | _MIN32)


_ROWS_PER_TILE = _ROWS // _NTILES
_CHUNKS_PER_ROW = _COLS // _CHUNK


_FLUSH_SMALL = 512


def _reduce_and_store(nbins, hist_v, red_v, dst):
    @plsc.parallel_loop(0, nbins // _LANES, unroll=4)
    def _(c):
        acc = hist_v[pl.ds(c * _LANES, _LANES)]
        for l in range(1, _LANES):
            acc = acc + hist_v[pl.ds(l * nbins + c * _LANES, _LANES)]
        red_v[pl.ds(c * _LANES, _LANES)] = acc
    pltpu.sync_copy(red_v, dst)


def _zero_hist(nbins, hist_v):
    @plsc.parallel_loop(0, nbins, unroll=8)
    def _(z):
        hist_v[pl.ds(z * _LANES, _LANES)] = jnp.zeros((_LANES,), jnp.int32)


def _hist_body(nbins, bucket_shift, prefix_shift, compact, x_hbm, prefix_hbm,
               *refs):
    if compact:
        (out_hbm, cand_hbm, cnt_hbm, buf0_v, buf1_v, pbuf_v, hist_v, red_v,
         cand_v, cntb_v, sem0, sem1) = refs
    else:
        out_hbm, buf0_v, buf1_v, pbuf_v, hist_v, red_v, sem0, sem1 = refs
        cand_hbm = cnt_hbm = cand_v = cntb_v = None
    wid = lax.axis_index("c") * 16 + lax.axis_index("s")
    row0 = wid * _ROWS_PER_TILE

    bufs = (buf0_v, buf1_v)
    sems = (sem0, sem1)

    def _chunk_src(ci):
        row = row0 + ci // _CHUNKS_PER_ROW
        col = (ci % _CHUNKS_PER_ROW) * _CHUNK
        return x_hbm.at[row, pl.ds(col, _CHUNK)]

    descs = [None, None]
    descs[0] = pltpu.async_copy(_chunk_src(0), bufs[0], sems[0])

    _zero_hist(nbins, hist_v)

    if prefix_shift is not None:
        pltpu.sync_copy(prefix_hbm, pbuf_v)
        pvec = pbuf_v[...]
    else:
        pvec = None
    liota = lax.iota(jnp.int32, _LANES)
    lane_base = liota * nbins
    ones = jnp.full((_LANES,), 1, jnp.int32)

    cnts_vec = jnp.zeros((_LANES,), jnp.int32)
    for ci in range(_NCHUNK):
        descs[ci % 2].wait()
        if ci + 1 < _NCHUNK:
            nxt = (ci + 1) % 2
            descs[nxt] = pltpu.async_copy(_chunk_src(ci + 1), bufs[nxt], sems[nxt])
        buf = bufs[ci % 2]

        if not compact:
            @plsc.parallel_loop(0, _CHUNK // _LANES, unroll=8)
            def _(j):
                key = _sortable_key(
                    plsc.bitcast(buf[pl.ds(j * _LANES, _LANES)], jnp.int32))
                bucket = lax.shift_right_logical(key, bucket_shift) & (nbins - 1)
                idx = lane_base + bucket
                if prefix_shift is None:
                    plsc.addupdate_scatter(hist_v, [idx], ones)
                else:
                    m = lax.shift_right_logical(key, prefix_shift) == pvec
                    plsc.addupdate_scatter(hist_v, [idx], ones, mask=m)
        else:
            @plsc.parallel_loop(0, _CHUNK // _LANES, unroll=8,
                                carry=jnp.int32(0))
            def cnt(j, cnt):
                key = _sortable_key(
                    plsc.bitcast(buf[pl.ds(j * _LANES, _LANES)], jnp.int32))
                m = lax.shift_right_logical(key, prefix_shift) == pvec
                plsc.store_compressed(cand_v.at[pl.ds(cnt, _LANES)], key,
                                      mask=m)
                return cnt + plsc.all_reduce_population_count(m)[0]

            def _cand_vreg(j, carry):
                key = cand_v[pl.ds(j * _LANES, _LANES)]
                bucket = lax.shift_right_logical(key, bucket_shift) & (nbins - 1)
                valid = (j * _LANES + liota) < cnt
                plsc.addupdate_scatter(hist_v, [lane_base + bucket], ones,
                                       mask=valid)
                return carry

            lax.fori_loop(0, lax.div(cnt + (_LANES - 1), _LANES), _cand_vreg,
                          jnp.int32(0))

            seg = wid * _SHARD + ci * _CHUNK

            @pl.when(jnp.logical_and(cnt > 0, cnt <= _FLUSH_SMALL))
            def _():
                pltpu.sync_copy(cand_v.at[pl.ds(0, _FLUSH_SMALL)],
                                cand_hbm.at[pl.ds(seg, _FLUSH_SMALL)])

            @pl.when(cnt > _FLUSH_SMALL)
            def _():
                pltpu.sync_copy(cand_v.at[pl.ds(0, _CHUNK)],
                                cand_hbm.at[pl.ds(seg, _CHUNK)])

            cnts_vec = jnp.where(liota == ci, cnt, cnts_vec)

    if compact:
        cntb_v[...] = cnts_vec
        pltpu.sync_copy(cntb_v, cnt_hbm.at[wid])
    _reduce_and_store(nbins, hist_v, red_v, out_hbm.at[wid])


def _cand_hist_body(nbins, prefix_shift, cand_hbm, cnt_hbm, prefix_hbm,
                    out_hbm, buf_v, stage_v, pbuf_v, cntb_v, hist_v, red_v,
                    *sems):
    wid = lax.axis_index("c") * 16 + lax.axis_index("s")

    descs = [
        pltpu.async_copy(
            cand_hbm.at[pl.ds(wid * _SHARD + ci * _CHUNK, _FLUSH_SMALL)],
            stage_v.at[pl.ds(ci * _FLUSH_SMALL, _FLUSH_SMALL)], sems[ci])
        for ci in range(_NCHUNK)
    ]
    pltpu.sync_copy(cnt_hbm.at[wid], cntb_v)
    _zero_hist(nbins, hist_v)
    pltpu.sync_copy(prefix_hbm, pbuf_v)
    pvec = pbuf_v[...]
    cnts = cntb_v[...]
    liota = lax.iota(jnp.int32, _LANES)
    lane_base = liota * nbins
    ones = jnp.full((_LANES,), 1, jnp.int32)

    def _seg_loop(src_v, src_off, cnt):
        def _vreg(j, carry):
            key = src_v[pl.ds(src_off + j * _LANES, _LANES)]
            bucket = key & (nbins - 1)
            valid = (j * _LANES + liota) < cnt
            m = jnp.logical_and(
                lax.shift_right_logical(key, prefix_shift) == pvec, valid)
            plsc.addupdate_scatter(hist_v, [lane_base + bucket], ones, mask=m)
            return carry

        lax.fori_loop(0, lax.div(cnt + (_LANES - 1), _LANES), _vreg,
                      jnp.int32(0))

    for ci in range(_NCHUNK):
        cnt = cnts[ci]
        descs[ci].wait()

        @pl.when(cnt <= _FLUSH_SMALL)
        def _():
            _seg_loop(stage_v, ci * _FLUSH_SMALL, cnt)

        @pl.when(cnt > _FLUSH_SMALL)
        def _():
            pltpu.sync_copy(
                cand_hbm.at[pl.ds(wid * _SHARD + ci * _CHUNK, _CHUNK)], buf_v)
            _seg_loop(buf_v, 0, cnt)

    _reduce_and_store(nbins, hist_v, red_v, out_hbm.at[wid])


def _make_hist_kernel(nbins, bucket_shift, prefix_shift, compact=False):
    mesh = plsc.VectorSubcoreMesh(core_axis_name="c", subcore_axis_name="s")
    if compact:
        out_type = [
            jax.ShapeDtypeStruct((_NTILES, nbins), jnp.int32),
            jax.ShapeDtypeStruct((_N,), jnp.int32),
            jax.ShapeDtypeStruct((_NTILES, _LANES), jnp.int32),
        ]
        extra = [pltpu.VMEM((_CHUNK + _LANES,), jnp.int32),
                 pltpu.VMEM((_LANES,), jnp.int32)]
    else:
        out_type = jax.ShapeDtypeStruct((_NTILES, nbins), jnp.int32)
        extra = []
    return functools.partial(
        pl.kernel,
        out_type=out_type,
        mesh=mesh,
        scratch_types=[
            pltpu.VMEM((_CHUNK,), jnp.float32),
            pltpu.VMEM((_CHUNK,), jnp.float32),
            pltpu.VMEM((_LANES,), jnp.int32),
            pltpu.VMEM((nbins * _LANES,), jnp.int32),
            pltpu.VMEM((nbins,), jnp.int32),
        ] + extra + [
            pltpu.SemaphoreType.DMA,
            pltpu.SemaphoreType.DMA,
        ],
        compiler_params=pltpu.CompilerParams(needs_layout_passes=False),
    )(functools.partial(_hist_body, nbins, bucket_shift, prefix_shift,
                        compact))


def _make_cand_hist_kernel(nbins, prefix_shift):
    mesh = plsc.VectorSubcoreMesh(core_axis_name="c", subcore_axis_name="s")
    return functools.partial(
        pl.kernel,
        out_type=jax.ShapeDtypeStruct((_NTILES, nbins), jnp.int32),
        mesh=mesh,
        scratch_types=[
            pltpu.VMEM((_CHUNK,), jnp.int32),
            pltpu.VMEM((_NCHUNK * _FLUSH_SMALL,), jnp.int32),
            pltpu.VMEM((_LANES,), jnp.int32),
            pltpu.VMEM((_LANES,), jnp.int32),
            pltpu.VMEM((nbins * _LANES,), jnp.int32),
            pltpu.VMEM((nbins,), jnp.int32),
        ] + [pltpu.SemaphoreType.DMA] * _NCHUNK,
        compiler_params=pltpu.CompilerParams(needs_layout_passes=False),
    )(functools.partial(_cand_hist_body, nbins, prefix_shift))


_hist_p1 = _make_hist_kernel(4096, 20, None)
_hist_p2 = _make_hist_kernel(4096, 8, 20, compact=True)
_hist_p3 = _make_cand_hist_kernel(256, 8)


def _pick_bin(tile_hists, nbins, k):
    del nbins
    hist = tile_hists.sum(axis=0)
    suffix = jnp.cumsum(hist[::-1])[::-1]
    b = jnp.sum((suffix >= k).astype(jnp.int32)) - 1
    k_next = k - (suffix[b] - hist[b])
    return b, k_next


def _mask_body(t_ref, x_ref, out_ref):
    ts = t_ref[0, 0] ^ _MIN32
    ubits = jnp.where(ts >= 0, ts, _MIN32 | (~ts))
    t = lax.bitcast_convert_type(ubits, jnp.float32)
    xv = x_ref[...]
    m = (xv >= t).astype(jnp.float32)
    out_ref[...] = xv + (m - xv)


def kernel(x):
    zeros16 = jnp.zeros((_LANES,), jnp.int32)

    h1 = _hist_p1(x, zeros16)
    b1, k2 = _pick_bin(h1, 4096, _K)
    h2, cand, cnts = _hist_p2(x, zeros16 + b1)
    b2, k3 = _pick_bin(h2, 4096, k2)
    h3 = _hist_p3(cand, cnts, zeros16 + ((b1 << 12) | b2))
    b3, _ = _pick_bin(h3, 256, k3)

    t_key = ((b1 << 20) | (b2 << 8) | b3).astype(jnp.int32).reshape(1, 1)

    block_rows = 32
    grid = _ROWS // block_rows
    out = pl.pallas_call(
        _mask_body,
        grid=(grid,),
        in_specs=[
            pl.BlockSpec(memory_space=pltpu.SMEM),
            pl.BlockSpec((block_rows, _COLS), lambda i: (i, 0)),
        ],
        out_specs=pl.BlockSpec((block_rows, _COLS), lambda i: (i, 0)),
        out_shape=jax.ShapeDtypeStruct((_ROWS, _COLS), jnp.float32),
    )(t_key, x)
    return out


# speedup vs baseline: 1.1074x; 1.0188x over previous
import functools

import jax
import jax.numpy as jnp
from jax import lax
from jax.experimental import pallas as pl
from jax.experimental.pallas import tpu as pltpu
from jax.experimental.pallas import tpu_sc as plsc

_ROWS, _COLS = 128, 32768
_N = _ROWS * _COLS
_K = int(_N * 0.9)
_MIN32 = -2147483648

_NTILES = 32
_SHARD = _N // _NTILES
_CHUNK = 16384
_NCHUNK = _SHARD // _CHUNK
_LANES = 16


def _sortable_key(v):
    return v ^ (lax.shift_right_arithmetic(v, 31) | _MIN32)


_ROWS_PER_TILE = _ROWS // _NTILES
_CHUNKS_PER_ROW = _COLS // _CHUNK


_FLUSH_SMALL = 512


def _reduce_and_store(nbins, hist_v, red_v, dst):
    @plsc.parallel_loop(0, nbins // _LANES, unroll=4)
    def _(c):
        acc = hist_v[pl.ds(c * _LANES, _LANES)]
        for l in range(1, _LANES):
            acc = acc + hist_v[pl.ds(l * nbins + c * _LANES, _LANES)]
        red_v[pl.ds(c * _LANES, _LANES)] = acc
    pltpu.sync_copy(red_v, dst)


def _zero_hist(nbins, hist_v):
    @plsc.parallel_loop(0, nbins, unroll=8)
    def _(z):
        hist_v[pl.ds(z * _LANES, _LANES)] = jnp.zeros((_LANES,), jnp.int32)


def _hist_body(nbins, bucket_shift, prefix_shift, compact, x_hbm, prefix_hbm,
               *refs):
    if compact:
        (out_hbm, cand_hbm, cnt_hbm, buf0_v, buf1_v, pbuf_v, hist_v, red_v,
         cand_v, cntb_v, sem0, sem1) = refs
    else:
        out_hbm, buf0_v, buf1_v, pbuf_v, hist_v, red_v, sem0, sem1 = refs
        cand_hbm = cnt_hbm = cand_v = cntb_v = None
    wid = lax.axis_index("c") * 16 + lax.axis_index("s")
    row0 = wid * _ROWS_PER_TILE

    bufs = (buf0_v, buf1_v)
    sems = (sem0, sem1)

    def _chunk_src(ci):
        row = row0 + ci // _CHUNKS_PER_ROW
        col = (ci % _CHUNKS_PER_ROW) * _CHUNK
        return x_hbm.at[row, pl.ds(col, _CHUNK)]

    descs = [None, None]
    descs[0] = pltpu.async_copy(_chunk_src(0), bufs[0], sems[0])

    _zero_hist(nbins, hist_v)

    if prefix_shift is not None:
        pltpu.sync_copy(prefix_hbm, pbuf_v)
        pvec = pbuf_v[...]
    else:
        pvec = None
    liota = lax.iota(jnp.int32, _LANES)
    lane_base = liota * nbins
    ones = jnp.full((_LANES,), 1, jnp.int32)

    cnts_vec = jnp.zeros((_LANES,), jnp.int32)
    for ci in range(_NCHUNK):
        descs[ci % 2].wait()
        if ci + 1 < _NCHUNK:
            nxt = (ci + 1) % 2
            descs[nxt] = pltpu.async_copy(_chunk_src(ci + 1), bufs[nxt], sems[nxt])
        buf = bufs[ci % 2]

        if not compact:
            @plsc.parallel_loop(0, _CHUNK // _LANES, unroll=8)
            def _(j):
                key = _sortable_key(
                    plsc.bitcast(buf[pl.ds(j * _LANES, _LANES)], jnp.int32))
                bucket = lax.shift_right_logical(key, bucket_shift) & (nbins - 1)
                idx = lane_base + bucket
                if prefix_shift is None:
                    plsc.addupdate_scatter(hist_v, [idx], ones)
                else:
                    m = lax.shift_right_logical(key, prefix_shift) == pvec
                    plsc.addupdate_scatter(hist_v, [idx], ones, mask=m)
        else:
            @plsc.parallel_loop(0, _CHUNK // _LANES, unroll=8,
                                carry=jnp.int32(0))
            def cnt(j, cnt):
                key = _sortable_key(
                    plsc.bitcast(buf[pl.ds(j * _LANES, _LANES)], jnp.int32))
                m = lax.shift_right_logical(key, prefix_shift) == pvec
                plsc.store_compressed(cand_v.at[pl.ds(cnt, _LANES)], key,
                                      mask=m)
                return cnt + plsc.all_reduce_population_count(m)[0]

            def _cand_vreg(j, carry):
                key = cand_v[pl.ds(j * _LANES, _LANES)]
                bucket = lax.shift_right_logical(key, bucket_shift) & (nbins - 1)
                valid = (j * _LANES + liota) < cnt
                plsc.addupdate_scatter(hist_v, [lane_base + bucket], ones,
                                       mask=valid)
                return carry

            lax.fori_loop(0, lax.div(cnt + (_LANES - 1), _LANES), _cand_vreg,
                          jnp.int32(0))

            seg = wid * _SHARD + ci * _CHUNK

            @pl.when(jnp.logical_and(cnt > 0, cnt <= _FLUSH_SMALL))
            def _():
                pltpu.sync_copy(cand_v.at[pl.ds(0, _FLUSH_SMALL)],
                                cand_hbm.at[pl.ds(seg, _FLUSH_SMALL)])

            @pl.when(cnt > _FLUSH_SMALL)
            def _():
                pltpu.sync_copy(cand_v.at[pl.ds(0, _CHUNK)],
                                cand_hbm.at[pl.ds(seg, _CHUNK)])

            cnts_vec = jnp.where(liota == ci, cnt, cnts_vec)

    if compact:
        cntb_v[...] = cnts_vec
        pltpu.sync_copy(cntb_v, cnt_hbm.at[wid])
    _reduce_and_store(nbins, hist_v, red_v, out_hbm.at[wid])


def _cand_hist_body(nbins, prefix_shift, cand_hbm, cnt_hbm, prefix_hbm,
                    out_hbm, buf_v, stage_v, pbuf_v, cntb_v, hist_v, red_v,
                    *sems):
    wid = lax.axis_index("c") * 16 + lax.axis_index("s")

    descs = [
        pltpu.async_copy(
            cand_hbm.at[pl.ds(wid * _SHARD + ci * _CHUNK, _FLUSH_SMALL)],
            stage_v.at[pl.ds(ci * _FLUSH_SMALL, _FLUSH_SMALL)], sems[ci])
        for ci in range(_NCHUNK)
    ]
    pltpu.sync_copy(cnt_hbm.at[wid], cntb_v)
    _zero_hist(nbins, hist_v)
    pltpu.sync_copy(prefix_hbm, pbuf_v)
    pvec = pbuf_v[...]
    cnts = cntb_v[...]
    liota = lax.iota(jnp.int32, _LANES)
    lane_base = liota * nbins
    ones = jnp.full((_LANES,), 1, jnp.int32)

    def _seg_loop(src_v, src_off, cnt):
        def _vreg(j, carry):
            key = src_v[pl.ds(src_off + j * _LANES, _LANES)]
            bucket = key & (nbins - 1)
            valid = (j * _LANES + liota) < cnt
            m = jnp.logical_and(
                lax.shift_right_logical(key, prefix_shift) == pvec, valid)
            plsc.addupdate_scatter(hist_v, [lane_base + bucket], ones, mask=m)
            return carry

        lax.fori_loop(0, lax.div(cnt + (_LANES - 1), _LANES), _vreg,
                      jnp.int32(0))

    for ci in range(_NCHUNK):
        cnt = cnts[ci]
        descs[ci].wait()

        @pl.when(cnt <= _FLUSH_SMALL)
        def _():
            _seg_loop(stage_v, ci * _FLUSH_SMALL, cnt)

        @pl.when(cnt > _FLUSH_SMALL)
        def _():
            pltpu.sync_copy(
                cand_hbm.at[pl.ds(wid * _SHARD + ci * _CHUNK, _CHUNK)], buf_v)
            _seg_loop(buf_v, 0, cnt)

    _reduce_and_store(nbins, hist_v, red_v, out_hbm.at[wid])


def _make_hist_kernel(nbins, bucket_shift, prefix_shift, compact=False):
    mesh = plsc.VectorSubcoreMesh(core_axis_name="c", subcore_axis_name="s")
    if compact:
        out_type = [
            jax.ShapeDtypeStruct((_NTILES, nbins), jnp.int32),
            jax.ShapeDtypeStruct((_N,), jnp.int32),
            jax.ShapeDtypeStruct((_NTILES, _LANES), jnp.int32),
        ]
        extra = [pltpu.VMEM((_CHUNK + _LANES,), jnp.int32),
                 pltpu.VMEM((_LANES,), jnp.int32)]
    else:
        out_type = jax.ShapeDtypeStruct((_NTILES, nbins), jnp.int32)
        extra = []
    return functools.partial(
        pl.kernel,
        out_type=out_type,
        mesh=mesh,
        scratch_types=[
            pltpu.VMEM((_CHUNK,), jnp.float32),
            pltpu.VMEM((_CHUNK,), jnp.float32),
            pltpu.VMEM((_LANES,), jnp.int32),
            pltpu.VMEM((nbins * _LANES,), jnp.int32),
            pltpu.VMEM((nbins,), jnp.int32),
        ] + extra + [
            pltpu.SemaphoreType.DMA,
            pltpu.SemaphoreType.DMA,
        ],
        compiler_params=pltpu.CompilerParams(needs_layout_passes=False),
    )(functools.partial(_hist_body, nbins, bucket_shift, prefix_shift,
                        compact))


def _make_cand_hist_kernel(nbins, prefix_shift):
    mesh = plsc.VectorSubcoreMesh(core_axis_name="c", subcore_axis_name="s")
    return functools.partial(
        pl.kernel,
        out_type=jax.ShapeDtypeStruct((_NTILES, nbins), jnp.int32),
        mesh=mesh,
        scratch_types=[
            pltpu.VMEM((_CHUNK,), jnp.int32),
            pltpu.VMEM((_NCHUNK * _FLUSH_SMALL,), jnp.int32),
            pltpu.VMEM((_LANES,), jnp.int32),
            pltpu.VMEM((_LANES,), jnp.int32),
            pltpu.VMEM((nbins * _LANES,), jnp.int32),
            pltpu.VMEM((nbins,), jnp.int32),
        ] + [pltpu.SemaphoreType.DMA] * _NCHUNK,
        compiler_params=pltpu.CompilerParams(needs_layout_passes=False),
    )(functools.partial(_cand_hist_body, nbins, prefix_shift))


_hist_p1 = _make_hist_kernel(4096, 20, None)
_hist_p2 = _make_hist_kernel(4096, 8, 20, compact=True)
_hist_p3 = _make_cand_hist_kernel(256, 8)


def _pick_bin(tile_hists, nbins, k):
    del nbins
    hist = tile_hists.sum(axis=0)
    suffix = jnp.cumsum(hist[::-1])[::-1]
    b = jnp.sum((suffix >= k).astype(jnp.int32)) - 1
    k_next = k - (suffix[b] - hist[b])
    return b, k_next


def _mask_body(t_ref, x_ref, out_ref):
    ts = t_ref[0, 0] ^ _MIN32
    ubits = jnp.where(ts >= 0, ts, _MIN32 | (~ts))
    t = lax.bitcast_convert_type(ubits, jnp.float32)
    xv = x_ref[...]
    m = (xv >= t).astype(jnp.float32)
    out_ref[...] = xv + (m - xv)


def kernel(x):
    zeros16 = jnp.zeros((_LANES,), jnp.int32)

    h1 = _hist_p1(x, zeros16)
    b1, k2 = _pick_bin(h1, 4096, _K)
    h2, cand, cnts = _hist_p2(x, zeros16 + b1)
    b2, k3 = _pick_bin(h2, 4096, k2)
    h3 = _hist_p3(cand, cnts, zeros16 + ((b1 << 12) | b2))
    b3, _ = _pick_bin(h3, 256, k3)

    t_key = ((b1 << 20) | (b2 << 8) | b3).astype(jnp.int32).reshape(1, 1)

    block_rows = 64
    grid = _ROWS // block_rows
    out = pl.pallas_call(
        _mask_body,
        grid=(grid,),
        in_specs=[
            pl.BlockSpec(memory_space=pltpu.SMEM),
            pl.BlockSpec((block_rows, _COLS), lambda i: (i, 0)),
        ],
        out_specs=pl.BlockSpec((block_rows, _COLS), lambda i: (i, 0)),
        out_shape=jax.ShapeDtypeStruct((_ROWS, _COLS), jnp.float32),
    )(t_key, x)
    return out
